# baseline (speedup 1.0000x reference)
import functools

import jax
import jax.numpy as jnp
from jax import lax
from jax.experimental import pallas as pl
from jax.experimental.pallas import tpu as pltpu
from jax.experimental.pallas import tpu_sc as plsc

_BATCH = 4096
_VOCAB = 100000
_DIM = 64
_NC = 2
_NS = 16
_NW = _NC * _NS
_RPW = _DIM // _NW
_LANES = 16
_H0 = 50048
_H1 = _VOCAB - _H0
_GROUPS = _BATCH // _LANES


@functools.partial(
    pl.kernel,
    mesh=plsc.VectorSubcoreMesh(core_axis_name="c", subcore_axis_name="s"),
    out_type=jax.ShapeDtypeStruct((_DIM, _BATCH), jnp.float32),
    scratch_types=[
        pltpu.VMEM((_BATCH,), jnp.int32),
        pltpu.VMEM((_BATCH,), jnp.int32),
        pltpu.VMEM((_H0 + 128,), jnp.float32),
        pltpu.VMEM((_H1,), jnp.float32),
        pltpu.VMEM((_BATCH,), jnp.float32),
        pltpu.VMEM((_BATCH,), jnp.float32),
        pltpu.VMEM((_BATCH,), jnp.float32),
        pltpu.VMEM((_BATCH,), jnp.float32),
        pltpu.VMEM((_BATCH,), jnp.float32),
        pltpu.SemaphoreType.DMA,
        pltpu.SemaphoreType.DMA,
        pltpu.SemaphoreType.DMA,
        pltpu.SemaphoreType.DMA,
    ],
    compiler_params=pltpu.CompilerParams(needs_layout_passes=False),
)
def _gather_add(xt_hbm, tablet_hbm, idx_hbm, outt_hbm,
                idx_v, i0_v, buf0, buf1, tmp_v, x_v0, x_v1, o_v0, o_v1,
                semA, semB, semX, semO):
    wid = lax.axis_index("s") * _NC + lax.axis_index("c")
    c0 = wid * _RPW
    x_vs = (x_v0, x_v1)
    o_vs = (o_v0, o_v1)

    def issue_low(row):
        return pltpu.async_copy(
            tablet_hbm.at[row, pl.ds(0, _H0 + 128)], buf0, semA)

    def issue_high(row):
        return pltpu.async_copy(
            tablet_hbm.at[row, pl.ds(_H0, _H1)], buf1, semB)

    cpB = issue_high(c0)
    cpA = issue_low(c0)
    cpXs = [pltpu.async_copy(xt_hbm.at[c0 + t], x_vs[t], semX)
            for t in range(_RPW)]
    pltpu.sync_copy(idx_hbm, idx_v)
    zeros = jnp.zeros((_LANES,), jnp.float32)

    def pass_high(g, carry):
        sl = pl.ds(g * _LANES, _LANES)
        iv = idx_v[sl]
        m = iv < _H0
        ih = jnp.maximum(iv, _H0) - _H0
        g1 = plsc.load_gather(buf1, [ih])
        tmp_v[sl] = jnp.where(m, 0.0, g1)
        i0_v[sl] = jnp.where(m, iv, _H0)
        return carry

    def make_pass_tail(x_v, o_v):
        def pass_tail(g, carry):
            sl = pl.ds(g * _LANES, _LANES)
            v0 = plsc.load_gather(buf0, [i0_v[sl]])
            o_v[sl] = x_v[sl] + (tmp_v[sl] + v0)
            return carry
        return pass_tail

    cpOs = []
    for t in range(_RPW):
        c = c0 + t
        cpB.wait()
        lax.fori_loop(0, _GROUPS, pass_high, 0)
        if t + 1 < _RPW:
            cpB = issue_high(c + 1)
        cpXs[t].wait()
        cpA.wait()
        buf0[pl.ds(_H0, _LANES)] = zeros
        lax.fori_loop(0, _GROUPS, make_pass_tail(x_vs[t], o_vs[t]), 0)
        if t + 1 < _RPW:
            cpA = issue_low(c + 1)
        cpOs.append(pltpu.async_copy(o_vs[t], outt_hbm.at[c], semO))
    for cp in cpOs:
        cp.wait()


def kernel(x, const, indices):
    out_t = _gather_add(x.T, const.T, indices.astype(jnp.int32))
    return out_t.T


# speedup vs baseline: 1.0184x; 1.0184x over previous
import functools

import jax
import jax.numpy as jnp
from jax import lax
from jax.experimental import pallas as pl
from jax.experimental.pallas import tpu as pltpu
from jax.experimental.pallas import tpu_sc as plsc

_BATCH = 4096
_VOCAB = 100000
_DIM = 64
_NC = 2
_NS = 16
_NW = _NC * _NS
_RPW = _DIM // _NW
_LANES = 16
_H0 = 50048
_H1 = _VOCAB - _H0
_GROUPS = _BATCH // _LANES


@functools.partial(
    pl.kernel,
    mesh=plsc.VectorSubcoreMesh(core_axis_name="c", subcore_axis_name="s"),
    out_type=jax.ShapeDtypeStruct((_DIM, _BATCH), jnp.float32),
    scratch_types=[
        pltpu.VMEM((_BATCH,), jnp.int32),
        pltpu.VMEM((_H0,), jnp.float32),
        pltpu.VMEM((_H1,), jnp.float32),
        pltpu.VMEM((_BATCH,), jnp.float32),
        pltpu.VMEM((_BATCH,), jnp.float32),
        pltpu.VMEM((_BATCH,), jnp.float32),
        pltpu.VMEM((_BATCH,), jnp.float32),
        pltpu.VMEM((_BATCH,), jnp.float32),
        pltpu.SemaphoreType.DMA,
        pltpu.SemaphoreType.DMA,
        pltpu.SemaphoreType.DMA,
        pltpu.SemaphoreType.DMA,
    ],
    compiler_params=pltpu.CompilerParams(needs_layout_passes=False),
)
def _gather_add(xt_hbm, tablet_hbm, idx_hbm, outt_hbm,
                idx_v, buf0, buf1, tmp_v, x_v0, x_v1, o_v0, o_v1,
                semA, semB, semX, semO):
    wid = lax.axis_index("s") * _NC + lax.axis_index("c")
    c0 = wid * _RPW
    x_vs = (x_v0, x_v1)
    o_vs = (o_v0, o_v1)

    cpA = pltpu.async_copy(tablet_hbm.at[c0, pl.ds(0, _H0)], buf0, semA)
    cpB = pltpu.async_copy(tablet_hbm.at[c0, pl.ds(_H0, _H1)], buf1, semB)
    cpXs = [pltpu.async_copy(xt_hbm.at[c0 + t], x_vs[t], semX)
            for t in range(_RPW)]
    pltpu.sync_copy(idx_hbm, idx_v)

    def pass_low(g, carry):
        sl = pl.ds(g * _LANES, _LANES)
        i0 = jnp.minimum(idx_v[sl], _H0 - 1)
        tmp_v[sl] = plsc.load_gather(buf0, [i0])
        return carry

    def make_pass_high(x_v, o_v):
        def pass_high(g, carry):
            sl = pl.ds(g * _LANES, _LANES)
            iv = idx_v[sl]
            i1 = jnp.minimum(jnp.maximum(iv, _H0) - _H0, _H1 - 1)
            v1 = plsc.load_gather(buf1, [i1])
            o_v[sl] = x_v[sl] + jnp.where(iv < _H0, tmp_v[sl], v1)
            return carry
        return pass_high

    cpOs = []
    for t in range(_RPW):
        c = c0 + t
        cpA.wait()
        lax.fori_loop(0, _GROUPS, pass_low, 0)
        if t + 1 < _RPW:
            cpA = pltpu.async_copy(
                tablet_hbm.at[c + 1, pl.ds(0, _H0)], buf0, semA)
        cpXs[t].wait()
        cpB.wait()
        lax.fori_loop(0, _GROUPS, make_pass_high(x_vs[t], o_vs[t]), 0)
        if t + 1 < _RPW:
            cpB = pltpu.async_copy(
                tablet_hbm.at[c + 1, pl.ds(_H0, _H1)], buf1, semB)
        cpOs.append(pltpu.async_copy(o_vs[t], outt_hbm.at[c], semO))
    for cp in cpOs:
        cp.wait()


def kernel(x, const, indices):
    out_t = _gather_add(x.T, const.T, indices.astype(jnp.int32))
    return out_t.T
